# baseline (speedup 1.0000x reference)
import functools

import jax
import jax.numpy as jnp
from jax import lax
from jax.experimental import pallas as pl
from jax.experimental.pallas import tpu as pltpu
from jax.experimental.pallas import tpu_sc as plsc

_N = 16777216
_BINS = 256
_NC, _NS, _L = 2, 16, 16
_NW = _NC * _NS
_PER_W = _N // _NW
_CHUNK = 32768
_NCHUNK = _PER_W // _CHUNK
_RSTRIDE = 264

_mesh = plsc.VectorSubcoreMesh(
    core_axis_name="c", subcore_axis_name="s",
    num_cores=_NC, num_subcores=_NS)


_MM_GRID = 8
_MM_BLK = _N // _MM_GRID


def _mm_body(x_ref, mn_ref, mx_ref, par_ref):
    i = pl.program_id(0)
    v = x_ref[...]
    bmn = jnp.min(v)
    bmx = jnp.max(v)

    @pl.when(i == 0)
    def _():
        mn_ref[0, 0] = bmn
        mx_ref[0, 0] = bmx

    @pl.when(i != 0)
    def _():
        mn_ref[0, 0] = jnp.minimum(mn_ref[0, 0], bmn)
        mx_ref[0, 0] = jnp.maximum(mx_ref[0, 0], bmx)

    @pl.when(i == _MM_GRID - 1)
    def _():
        lo = jnp.trunc(mn_ref[0, 0])
        span = jnp.trunc(mx_ref[0, 0]) - lo
        safe = jnp.where(span == 0, jnp.float32(1.0), span)
        scale = jnp.float32(_BINS) / safe
        par_ref[...] = jnp.concatenate(
            [jnp.full((_L,), lo), jnp.full((_L,), scale)])


_minmax = pl.pallas_call(
    _mm_body,
    grid=(_MM_GRID,),
    in_specs=[pl.BlockSpec((_MM_BLK // 128, 128), lambda i: (i, 0))],
    out_specs=[pl.BlockSpec(memory_space=pltpu.SMEM),
               pl.BlockSpec(memory_space=pltpu.SMEM),
               pl.BlockSpec((2 * _L,), lambda i: (0,))],
    out_shape=[jax.ShapeDtypeStruct((1, 1), jnp.float32),
               jax.ShapeDtypeStruct((1, 1), jnp.float32),
               jax.ShapeDtypeStruct((2 * _L,), jnp.float32)],
)


def _hist_body(x_hbm, par_hbm, out_hbm, xout_hbm,
               buf0, buf1, buf2, par_v, hist_v, out_v,
               rs0, rs1, rs2, ws0, ws1, ws2):
    wid = lax.axis_index("s") * _NC + lax.axis_index("c")
    base = wid * _PER_W

    pltpu.sync_copy(par_hbm, par_v)
    lo = par_v[pl.ds(0, _L)]
    scale = par_v[pl.ds(_L, _L)]

    lane_base = lax.iota(jnp.int32, _L) * _RSTRIDE + 1
    ones = jnp.ones((_L,), jnp.float32)
    zeros = jnp.zeros((_L,), jnp.float32)

    def _zero(j, _):
        hist_v[pl.ds(j * _L, _L)] = zeros
        return 0
    lax.fori_loop(0, (_L * _RSTRIDE) // _L, _zero, 0)

    rsems = (rs0, rs1, rs2)
    wsems = (ws0, ws1, ws2)
    bufs = (buf0, buf1, buf2)
    rhdl = [None, None, None]
    whdl = [None, None, None]
    for p in range(2):
        rhdl[p] = pltpu.async_copy(
            x_hbm.at[pl.ds(base + p * _CHUNK, _CHUNK)], bufs[p], rsems[p])
    for c in range(_NCHUNK):
        b = c % 3
        rhdl[b].wait()
        whdl[b] = pltpu.async_copy(
            bufs[b], xout_hbm.at[pl.ds(base + c * _CHUNK, _CHUNK)], wsems[b])
        if c + 2 < _NCHUNK:
            nb = (c + 2) % 3
            if whdl[nb] is not None:
                whdl[nb].wait()
                whdl[nb] = None
            rhdl[nb] = pltpu.async_copy(
                x_hbm.at[pl.ds(base + (c + 2) * _CHUNK, _CHUNK)],
                bufs[nb], rsems[nb])
        bb = bufs[b]

        @plsc.parallel_loop(0, _CHUNK, _L, unroll=2)
        def _vstep(i):
            v = bb[pl.ds(i, _L)]
            t = (v - lo) * scale
            ii = t.astype(jnp.int32)
            idx = jnp.minimum(jnp.maximum(ii, -1), _BINS)
            plsc.addupdate_scatter(hist_v, [lane_base + idx], ones)

    for b in range(3):
        if whdl[b] is not None:
            whdl[b].wait()

    def _rcol(j, _):
        def _rrow(r, acc):
            return acc + hist_v[pl.ds(r * _RSTRIDE + 1 + j * _L, _L)]
        acc = lax.fori_loop(0, _L, _rrow, zeros)
        out_v[pl.ds(j * _L, _L)] = acc
        return 0
    lax.fori_loop(0, _BINS // _L, _rcol, 0)

    pltpu.sync_copy(out_v, out_hbm.at[wid])


_hist = functools.partial(
    pl.kernel,
    out_type=[jax.ShapeDtypeStruct((_NW, _BINS), jnp.float32),
              jax.ShapeDtypeStruct((_N,), jnp.float32)],
    mesh=_mesh,
    compiler_params=pltpu.CompilerParams(needs_layout_passes=False),
    scratch_types=[
        pltpu.VMEM((_CHUNK,), jnp.float32),
        pltpu.VMEM((_CHUNK,), jnp.float32),
        pltpu.VMEM((_CHUNK,), jnp.float32),
        pltpu.VMEM((2 * _L,), jnp.float32),
        pltpu.VMEM((_L * _RSTRIDE,), jnp.float32),
        pltpu.VMEM((_BINS,), jnp.float32),
        pltpu.SemaphoreType.DMA,
        pltpu.SemaphoreType.DMA,
        pltpu.SemaphoreType.DMA,
        pltpu.SemaphoreType.DMA,
        pltpu.SemaphoreType.DMA,
        pltpu.SemaphoreType.DMA,
    ],
)(_hist_body)


def kernel(x_orig):
    x = lax.stop_gradient(x_orig)
    mn, mx, params = _minmax(x.reshape(_N // 128, 128))
    mn_s = mn[0, 0]
    mx_s = mx[0, 0]
    parts, x_out = _hist(x, params)
    histogram = jnp.sum(parts, axis=0)
    return x_out, histogram, mn_s, mx_s


# speedup vs baseline: 1.0589x; 1.0589x over previous
import functools

import jax
import jax.numpy as jnp
from jax import lax
from jax.experimental import pallas as pl
from jax.experimental.pallas import tpu as pltpu
from jax.experimental.pallas import tpu_sc as plsc

_N = 16777216
_BINS = 256
_NC, _NS, _L = 2, 16, 16
_NW = _NC * _NS
_PER_W = _N // _NW
_CHUNK = 16384
_NCHUNK = _PER_W // _CHUNK
_RSTRIDE = 264

_mesh = plsc.VectorSubcoreMesh(
    core_axis_name="c", subcore_axis_name="s",
    num_cores=_NC, num_subcores=_NS)


_MM_GRID = 8
_MM_BLK = _N // _MM_GRID


def _mm_body(x_ref, mn_ref, mx_ref, par_ref):
    i = pl.program_id(0)
    v = x_ref[...]
    bmn = jnp.min(v)
    bmx = jnp.max(v)

    @pl.when(i == 0)
    def _():
        mn_ref[0, 0] = bmn
        mx_ref[0, 0] = bmx

    @pl.when(i != 0)
    def _():
        mn_ref[0, 0] = jnp.minimum(mn_ref[0, 0], bmn)
        mx_ref[0, 0] = jnp.maximum(mx_ref[0, 0], bmx)

    @pl.when(i == _MM_GRID - 1)
    def _():
        lo = jnp.trunc(mn_ref[0, 0])
        span = jnp.trunc(mx_ref[0, 0]) - lo
        safe = jnp.where(span == 0, jnp.float32(1.0), span)
        scale = jnp.float32(_BINS) / safe
        par_ref[...] = jnp.concatenate(
            [jnp.full((_L,), lo), jnp.full((_L,), scale)])


_minmax = pl.pallas_call(
    _mm_body,
    grid=(_MM_GRID,),
    in_specs=[pl.BlockSpec((_MM_BLK // 128, 128), lambda i: (i, 0))],
    out_specs=[pl.BlockSpec(memory_space=pltpu.SMEM),
               pl.BlockSpec(memory_space=pltpu.SMEM),
               pl.BlockSpec((2 * _L,), lambda i: (0,))],
    out_shape=[jax.ShapeDtypeStruct((1, 1), jnp.float32),
               jax.ShapeDtypeStruct((1, 1), jnp.float32),
               jax.ShapeDtypeStruct((2 * _L,), jnp.float32)],
)


def _hist_body(x_hbm, par_hbm, out_hbm, xout_hbm,
               buf0, buf1, buf2, par_v, hist_v, out_v,
               rs0, rs1, rs2, ws0, ws1, ws2):
    wid = lax.axis_index("s") * _NC + lax.axis_index("c")
    base = wid * _PER_W

    pltpu.sync_copy(par_hbm, par_v)
    lo = par_v[pl.ds(0, _L)]
    scale = par_v[pl.ds(_L, _L)]

    lane_base = lax.iota(jnp.int32, _L) * _RSTRIDE + 1
    ones = jnp.ones((_L,), jnp.float32)
    zeros = jnp.zeros((_L,), jnp.float32)

    def _zero(j, _):
        hist_v[pl.ds(j * _L, _L)] = zeros
        return 0
    lax.fori_loop(0, (_L * _RSTRIDE) // _L, _zero, 0)

    rsems = (rs0, rs1, rs2)
    wsems = (ws0, ws1, ws2)
    bufs = (buf0, buf1, buf2)
    rhdl = [None, None, None]
    whdl = [None, None, None]
    for p in range(2):
        rhdl[p] = pltpu.async_copy(
            x_hbm.at[pl.ds(base + p * _CHUNK, _CHUNK)], bufs[p], rsems[p])
    for c in range(_NCHUNK):
        b = c % 3
        rhdl[b].wait()
        whdl[b] = pltpu.async_copy(
            bufs[b], xout_hbm.at[pl.ds(base + c * _CHUNK, _CHUNK)], wsems[b])
        if c + 2 < _NCHUNK:
            nb = (c + 2) % 3
            if whdl[nb] is not None:
                whdl[nb].wait()
                whdl[nb] = None
            rhdl[nb] = pltpu.async_copy(
                x_hbm.at[pl.ds(base + (c + 2) * _CHUNK, _CHUNK)],
                bufs[nb], rsems[nb])
        bb = bufs[b]

        @plsc.parallel_loop(0, _CHUNK, _L, unroll=4)
        def _vstep(i):
            v = bb[pl.ds(i, _L)]
            t = (v - lo) * scale
            ii = t.astype(jnp.int32)
            idx = jnp.minimum(jnp.maximum(ii, -1), _BINS)
            plsc.addupdate_scatter(hist_v, [lane_base + idx], ones)

    for b in range(3):
        if whdl[b] is not None:
            whdl[b].wait()

    def _rcol(j, _):
        def _rrow(r, acc):
            return acc + hist_v[pl.ds(r * _RSTRIDE + 1 + j * _L, _L)]
        acc = lax.fori_loop(0, _L, _rrow, zeros)
        out_v[pl.ds(j * _L, _L)] = acc
        return 0
    lax.fori_loop(0, _BINS // _L, _rcol, 0)

    pltpu.sync_copy(out_v, out_hbm.at[wid])


_hist = functools.partial(
    pl.kernel,
    out_type=[jax.ShapeDtypeStruct((_NW, _BINS), jnp.float32),
              jax.ShapeDtypeStruct((_N,), jnp.float32)],
    mesh=_mesh,
    compiler_params=pltpu.CompilerParams(needs_layout_passes=False),
    scratch_types=[
        pltpu.VMEM((_CHUNK,), jnp.float32),
        pltpu.VMEM((_CHUNK,), jnp.float32),
        pltpu.VMEM((_CHUNK,), jnp.float32),
        pltpu.VMEM((2 * _L,), jnp.float32),
        pltpu.VMEM((_L * _RSTRIDE,), jnp.float32),
        pltpu.VMEM((_BINS,), jnp.float32),
        pltpu.SemaphoreType.DMA,
        pltpu.SemaphoreType.DMA,
        pltpu.SemaphoreType.DMA,
        pltpu.SemaphoreType.DMA,
        pltpu.SemaphoreType.DMA,
        pltpu.SemaphoreType.DMA,
    ],
)(_hist_body)


def kernel(x_orig):
    x = lax.stop_gradient(x_orig)
    mn, mx, params = _minmax(x.reshape(_N // 128, 128))
    mn_s = mn[0, 0]
    mx_s = mx[0, 0]
    parts, x_out = _hist(x, params)
    histogram = jnp.sum(parts, axis=0)
    return x_out, histogram, mn_s, mx_s
